# baseline (speedup 1.0000x reference)
import jax
import jax.numpy as jnp
from jax import lax
from jax.experimental import pallas as pl
from jax.experimental.pallas import tpu as pltpu
from jax.experimental.pallas import tpu_sc as plsc

HID = 64
NC, NS = 2, 16
NW = NC * NS
S = 50
CHUNK = 128
NTC = 16384 // CHUNK
TCW = NTC // NW
NG = S * TCW


def _gather_body(seq_hbm, table_hbm, out_hbm, idx_v, rows_v, t_v, gsem, osem):
    wid = lax.axis_index("s") * NC + lax.axis_index("c")
    pltpu.sync_copy(seq_hbm.at[:, pl.ds(wid * TCW, TCW)], idx_v)

    lanes = jax.lax.broadcasted_iota(jnp.int32, (16,), 0)

    def g_copy(b, g):
        s, tcl = g // TCW, g % TCW
        return pltpu.make_async_copy(
            table_hbm.at[idx_v.at[s, tcl]], rows_v.at[b], gsem.at[b])

    def o_copy(b, g):
        s, tcl = g // TCW, g % TCW
        return pltpu.make_async_copy(
            t_v.at[b], out_hbm.at[s, :, wid * TCW + tcl], osem.at[b])

    g_copy(0, 0).start()

    def body(g, carry):
        b = g % 2
        nb = 1 - b

        @pl.when(g + 1 < NG)
        def _fire_next():
            g_copy(nb, g + 1).start()

        g_copy(b, g).wait()

        @pl.when(g >= 2)
        def _drain_old():
            o_copy(b, g - 2).wait()

        bv = jnp.zeros((16,), jnp.int32) + b
        for l in range(8):
            ridx = lanes + (l * 16)
            for tr in range(8):
                for c8 in range(8):
                    v = plsc.load_gather(
                        rows_v, [bv, ridx, jnp.zeros((16,), jnp.int32) + (tr * 8 + c8)])
                    plsc.store_scatter(
                        t_v,
                        [bv, jnp.zeros((16,), jnp.int32) + tr,
                         jnp.zeros((16,), jnp.int32) + c8, ridx],
                        v)

        o_copy(b, g).start()
        return carry

    lax.fori_loop(0, NG, body, 0)
    o_copy((NG - 2) % 2, NG - 2).wait()
    o_copy((NG - 1) % 2, NG - 1).wait()


def kernel(seq, table):
    seq3 = seq.T.reshape(S, NTC, CHUNK)
    out5 = pl.kernel(
        _gather_body,
        out_type=jax.ShapeDtypeStruct((S, HID // 8, NTC, 8, CHUNK), jnp.float32),
        mesh=plsc.VectorSubcoreMesh(core_axis_name="c", subcore_axis_name="s"),
        scratch_types=[
            pltpu.VMEM((S, TCW, CHUNK), jnp.int32),
            pltpu.VMEM((2, CHUNK, HID), jnp.float32),
            pltpu.VMEM((2, HID // 8, 8, CHUNK), jnp.float32),
            pltpu.SemaphoreType.DMA((2,)),
            pltpu.SemaphoreType.DMA((2,)),
        ],
        compiler_params=pltpu.CompilerParams(use_tc_tiling_on_sc=False, needs_layout_passes=False),
    )(seq3, table)
    return out5.transpose(2, 4, 0, 1, 3).reshape(16384, S, HID)
